# baseline (speedup 1.0000x reference)
import functools
import math

import jax
import jax.numpy as jnp
from jax import lax
from jax.experimental import pallas as pl
from jax.experimental.pallas import tpu as pltpu
from jax.experimental.pallas import tpu_sc as plsc

_NC = 2
_NS = 16
_NW = _NC * _NS
_CHUNK = 128
_LAG = 48


def _sc_gather_t(emb_t, sx_t, fc_flat, xoff_t):
    fd, v = emb_t.shape
    f, cb, _ = sx_t.shape
    b = cb * _CHUNK
    d = fd // f
    rows_per_tile = fd // _NW
    do_fc = fc_flat is not None
    fc_per_tile = (xoff_t.shape[0] // _NW) if do_fc else 0

    mesh = plsc.VectorSubcoreMesh(core_axis_name="c", subcore_axis_name="s")

    out_type = [jax.ShapeDtypeStruct((fd, b), jnp.float32)]
    scratch = [
        pltpu.VMEM((2, cb, _CHUNK), jnp.int32),
        pltpu.VMEM((rows_per_tile, b), jnp.float32),
        pltpu.SemaphoreType.DMA,
        pltpu.SemaphoreType.DMA,
    ]
    if do_fc:
        out_type.append(
            jax.ShapeDtypeStruct((xoff_t.shape[0], _CHUNK), jnp.float32))
        scratch += [
            pltpu.VMEM((fc_per_tile, _CHUNK), jnp.int32),
            pltpu.VMEM((fc_per_tile, _CHUNK), jnp.float32),
            pltpu.SemaphoreType.DMA,
        ]

    @functools.partial(
        pl.kernel,
        mesh=mesh,
        out_type=out_type,
        scratch_types=scratch,
        compiler_params=pltpu.CompilerParams(use_tc_tiling_on_sc=False),
    )
    def sc_kernel(*args):
        if do_fc:
            (emb_hbm, sx_hbm, fc_hbm, xoff_hbm, et_out, fcv_out,
             idx_v, rows_v, sem, sem2, fcx_v, fcv_v, fsem) = args
        else:
            (emb_hbm, sx_hbm, et_out, idx_v, rows_v, sem, sem2) = args
        wid = lax.axis_index("s") * _NC + lax.axis_index("c")
        row0 = wid * rows_per_tile
        f_lo = jnp.minimum(row0 // d, f - 2)
        pltpu.sync_copy(sx_hbm.at[pl.ds(f_lo, 2)], idx_v)

        n_emb = rows_per_tile * cb

        def emb_copy(n, s):
            r = n // cb
            ck = n % cb
            f_rel = (row0 + r) // d - f_lo
            return pltpu.make_async_copy(
                emb_hbm.at[row0 + r].at[idx_v.at[f_rel, ck]],
                rows_v.at[r, pl.ds(ck * _CHUNK, _CHUNK)], s)

        def emb_body(m, carry):
            emb_copy(2 * m, sem).start()
            emb_copy(2 * m + 1, sem2).start()

            @pl.when(2 * m >= _LAG)
            def _():
                emb_copy(2 * m - _LAG, sem).wait()
                emb_copy(2 * m + 1 - _LAG, sem2).wait()

            return carry

        lax.fori_loop(0, n_emb // 2, emb_body, 0)

        def drain_body(n, carry):
            emb_copy(n_emb - _LAG + 2 * n, sem).wait()
            emb_copy(n_emb - _LAG + 2 * n + 1, sem2).wait()
            return carry

        lax.fori_loop(0, _LAG // 2, drain_body, 0)

        if do_fc:
            pltpu.sync_copy(
                xoff_hbm.at[pl.ds(wid * fc_per_tile, fc_per_tile)], fcx_v)

            def fc_copy(j):
                return pltpu.make_async_copy(
                    fc_hbm.at[0].at[fcx_v.at[j]], fcv_v.at[j], fsem)

            def fc_body(j, carry):
                fc_copy(j).start()

                @pl.when(j >= 4)
                def _():
                    fc_copy(j - 4).wait()

                return carry

            lax.fori_loop(0, fc_per_tile, fc_body, 0)

            def fc_drain(j, carry):
                fc_copy(fc_per_tile - 4 + j).wait()
                return carry

            lax.fori_loop(0, 4, fc_drain, 0)

        pltpu.sync_copy(rows_v, et_out.at[pl.ds(row0, rows_per_tile)])
        if do_fc:
            pltpu.sync_copy(
                fcv_v, fcv_out.at[pl.ds(wid * fc_per_tile, fc_per_tile)])

    if do_fc:
        return sc_kernel(emb_t, sx_t, fc_flat, xoff_t)
    res = sc_kernel(emb_t, sx_t)
    return res[0] if isinstance(res, (list, tuple)) else res


def _tc_forward_t(e0, e1, dx_t, fcv_t, bias11,
                  W1, b1, g1, be1, W2, b2, g2, be2, W3, b3, g3, be3, Wo, bo11,
                  eps):
    fdh, b_total = e0.shape
    fd = 2 * fdh
    nd = dx_t.shape[0]
    f = fcv_t.shape[0]
    d = fd // f
    d_in = fd + nd
    inv = 1.0 / math.sqrt(1.0 + eps)
    bB = 1024
    grid = (b_total // bB,)

    dot0 = functools.partial(
        lax.dot_general,
        dimension_numbers=(((0,), (0,)), ((), ())),
        preferred_element_type=jnp.float32)

    def body(e0_ref, e1_ref, dx_ref, fcv_ref, bias_ref,
             w1_ref, b1_ref, g1_ref, be1_ref,
             w2_ref, b2_ref, g2_ref, be2_ref,
             w3_ref, b3_ref, g3_ref, be3_ref,
             wo_ref, bo_ref, out_ref):
        ea = e0_ref[...]
        eb = e1_ref[...]
        r = lax.broadcasted_iota(jnp.int32, (fdh, d), 0) % d
        c = lax.broadcasted_iota(jnp.int32, (fdh, d), 1)
        sel = (r == c).astype(jnp.float32)
        s = dot0(sel, ea) + dot0(sel, eb)
        ss = dot0(sel, ea * ea) + dot0(sel, eb * eb)
        fm = 0.5 * jnp.sum(s * s - ss, axis=0, keepdims=True)
        lin = jnp.sum(fcv_ref[...], axis=0, keepdims=True) + bias_ref[0, 0]
        h = (dot0(w1_ref[:fdh, :], ea)
             + dot0(w1_ref[fdh:fd, :], eb)
             + dot0(w1_ref[fd:d_in, :], dx_ref[...])
             + b1_ref[...])
        h = jnp.maximum(h * (g1_ref[...] * inv) + be1_ref[...], 0.0)
        h = dot0(w2_ref[...], h) + b2_ref[...]
        h = jnp.maximum(h * (g2_ref[...] * inv) + be2_ref[...], 0.0)
        h = dot0(w3_ref[...], h) + b3_ref[...]
        h = jnp.maximum(h * (g3_ref[...] * inv) + be3_ref[...], 0.0)
        deep = dot0(wo_ref[...], h) + bo_ref[0, 0]
        y = lin + fm + deep
        out_ref[...] = 1.0 / (1.0 + jnp.exp(-y))

    full = lambda a: pl.BlockSpec(a.shape, lambda i: (0,) * a.ndim)
    out = pl.pallas_call(
        body,
        grid=grid,
        in_specs=[
            pl.BlockSpec((fdh, bB), lambda i: (0, i)),
            pl.BlockSpec((fdh, bB), lambda i: (0, i)),
            pl.BlockSpec((nd, bB), lambda i: (0, i)),
            pl.BlockSpec((f, bB), lambda i: (0, i)),
            full(bias11),
            full(W1), full(b1), full(g1), full(be1),
            full(W2), full(b2), full(g2), full(be2),
            full(W3), full(b3), full(g3), full(be3),
            full(Wo), full(bo11),
        ],
        out_specs=pl.BlockSpec((1, bB), lambda i: (0, i)),
        out_shape=jax.ShapeDtypeStruct((1, b_total), jnp.float32),
    )(e0, e1, dx_t, fcv_t, bias11,
      W1, b1, g1, be1, W2, b2, g2, be2, W3, b3, g3, be3, Wo, bo11)
    return out.reshape(b_total)


def kernel(sparse_x, dense_x, emb, fc_w, bias,
           W1, b1, g1, be1, W2, b2, g2, be2, W3, b3, g3, be3, Wo, bo):
    b, f = sparse_x.shape
    _, v, d = emb.shape
    cb = b // _CHUNK
    fh = f // 2

    emb_t = emb.transpose(0, 2, 1).reshape(f * d, v)
    fc_flat = fc_w.T
    sx_t = sparse_x.T.astype(jnp.int32)
    offs = (jnp.arange(f, dtype=jnp.int32) * v)[:, None]
    xoff_t = (sx_t + offs).reshape(f * cb, _CHUNK)
    sx_t3 = sx_t.reshape(f, cb, _CHUNK)

    e0, fcv = _sc_gather_t(emb_t[:fh * d], sx_t3[:fh], fc_flat, xoff_t)
    e1 = _sc_gather_t(emb_t[fh * d:], sx_t3[fh:], None, None)

    fcv_t = fcv.reshape(f, b)
    dx_t = dense_x.T
    col = lambda a: a.reshape(-1, 1)
    out = _tc_forward_t(e0, e1, dx_t, fcv_t, bias.reshape(1, 1),
                        W1, col(b1), col(g1), col(be1),
                        W2, col(b2), col(g2), col(be2),
                        W3, col(b3), col(g3), col(be3),
                        Wo, bo.reshape(1, 1), 1e-5)
    return out


# speedup vs baseline: 1.2074x; 1.2074x over previous
import functools
import math

import jax
import jax.numpy as jnp
from jax import lax
from jax.experimental import pallas as pl
from jax.experimental.pallas import tpu as pltpu
from jax.experimental.pallas import tpu_sc as plsc

_NC = 2
_NS = 16
_NW = _NC * _NS
_CHUNK = 128
_LAG = 48


def _sc_gather_t(emb_t, fc_flat, sx_t, xoff_t):
    fd, v = emb_t.shape
    f, cb, _ = sx_t.shape
    b = cb * _CHUNK
    d = fd // f
    rows_per_tile = fd // _NW
    fc_per_tile = (f * cb) // _NW

    mesh = plsc.VectorSubcoreMesh(core_axis_name="c", subcore_axis_name="s")

    @functools.partial(
        pl.kernel,
        mesh=mesh,
        out_type=[
            jax.ShapeDtypeStruct((fd, b), jnp.float32),
            jax.ShapeDtypeStruct((f * cb, _CHUNK), jnp.float32),
        ],
        scratch_types=[
            pltpu.VMEM((2, cb, _CHUNK), jnp.int32),
            pltpu.VMEM((fc_per_tile, _CHUNK), jnp.int32),
            pltpu.VMEM((rows_per_tile, b), jnp.float32),
            pltpu.VMEM((fc_per_tile, _CHUNK), jnp.float32),
            pltpu.SemaphoreType.DMA,
            pltpu.SemaphoreType.DMA,
            pltpu.SemaphoreType.DMA,
        ],
        compiler_params=pltpu.CompilerParams(use_tc_tiling_on_sc=False),
    )
    def sc_kernel(emb_hbm, fc_hbm, sx_hbm, xoff_hbm, et_out, fcv_out,
                  idx_v, fcx_v, rows_v, fcv_v, sem, sem2, fsem):
        wid = lax.axis_index("s") * _NC + lax.axis_index("c")
        row0 = wid * rows_per_tile
        f_lo = jnp.minimum(row0 // d, f - 2)
        pltpu.sync_copy(sx_hbm.at[pl.ds(f_lo, 2)], idx_v)
        pltpu.sync_copy(xoff_hbm.at[pl.ds(wid * fc_per_tile, fc_per_tile)],
                        fcx_v)

        n_emb = rows_per_tile * cb

        def emb_copy(n, s):
            r = n // cb
            ck = n % cb
            f_rel = (row0 + r) // d - f_lo
            return pltpu.make_async_copy(
                emb_hbm.at[row0 + r].at[idx_v.at[f_rel, ck]],
                rows_v.at[r, pl.ds(ck * _CHUNK, _CHUNK)], s)

        def emb_body(m, carry):
            emb_copy(2 * m, sem).start()
            emb_copy(2 * m + 1, sem2).start()

            @pl.when(2 * m >= _LAG)
            def _():
                emb_copy(2 * m - _LAG, sem).wait()
                emb_copy(2 * m + 1 - _LAG, sem2).wait()

            return carry

        lax.fori_loop(0, n_emb // 2, emb_body, 0)

        def drain_body(n, carry):
            emb_copy(n_emb - _LAG + 2 * n, sem).wait()
            emb_copy(n_emb - _LAG + 2 * n + 1, sem2).wait()
            return carry

        lax.fori_loop(0, _LAG // 2, drain_body, 0)

        def fc_copy(j):
            return pltpu.make_async_copy(
                fc_hbm.at[0].at[fcx_v.at[j]], fcv_v.at[j], fsem)

        def fc_body(j, carry):
            fc_copy(j).start()

            @pl.when(j >= 4)
            def _():
                fc_copy(j - 4).wait()

            return carry

        lax.fori_loop(0, fc_per_tile, fc_body, 0)

        def fc_drain(j, carry):
            fc_copy(fc_per_tile - 4 + j).wait()
            return carry

        lax.fori_loop(0, 4, fc_drain, 0)

        pltpu.sync_copy(rows_v, et_out.at[pl.ds(row0, rows_per_tile)])
        pltpu.sync_copy(fcv_v,
                        fcv_out.at[pl.ds(wid * fc_per_tile, fc_per_tile)])

    return sc_kernel(emb_t, fc_flat, sx_t, xoff_t)


def _tc_forward_t(e_t, dx_t, fcv_t, bias11,
                  W1, b1, g1, be1, W2, b2, g2, be2, W3, b3, g3, be3, Wo, bo11,
                  eps):
    fd, b_total = e_t.shape
    nd = dx_t.shape[0]
    f = fcv_t.shape[0]
    d = fd // f
    d_in = fd + nd
    inv = 1.0 / math.sqrt(1.0 + eps)
    bB = 1024
    grid = (b_total // bB,)

    dot0 = functools.partial(
        lax.dot_general,
        dimension_numbers=(((0,), (0,)), ((), ())),
        preferred_element_type=jnp.float32)

    def body(e_ref, dx_ref, fcv_ref, bias_ref,
             w1_ref, b1_ref, g1_ref, be1_ref,
             w2_ref, b2_ref, g2_ref, be2_ref,
             w3_ref, b3_ref, g3_ref, be3_ref,
             wo_ref, bo_ref, out_ref):
        e = e_ref[...]
        r = lax.broadcasted_iota(jnp.int32, (fd, d), 0) % d
        c = lax.broadcasted_iota(jnp.int32, (fd, d), 1)
        sel = (r == c).astype(jnp.float32)
        s = dot0(sel, e)
        ss = dot0(sel, e * e)
        fm = 0.5 * jnp.sum(s * s - ss, axis=0, keepdims=True)
        lin = jnp.sum(fcv_ref[...], axis=0, keepdims=True) + bias_ref[0, 0]
        h = (dot0(w1_ref[:fd, :], e)
             + dot0(w1_ref[fd:d_in, :], dx_ref[...])
             + b1_ref[...])
        h = jnp.maximum(h * (g1_ref[...] * inv) + be1_ref[...], 0.0)
        h = dot0(w2_ref[...], h) + b2_ref[...]
        h = jnp.maximum(h * (g2_ref[...] * inv) + be2_ref[...], 0.0)
        h = dot0(w3_ref[...], h) + b3_ref[...]
        h = jnp.maximum(h * (g3_ref[...] * inv) + be3_ref[...], 0.0)
        deep = dot0(wo_ref[...], h) + bo_ref[0, 0]
        y = lin + fm + deep
        out_ref[...] = 1.0 / (1.0 + jnp.exp(-y))

    full = lambda a: pl.BlockSpec(a.shape, lambda i: (0,) * a.ndim)
    out = pl.pallas_call(
        body,
        grid=grid,
        in_specs=[
            pl.BlockSpec((fd, bB), lambda i: (0, i)),
            pl.BlockSpec((nd, bB), lambda i: (0, i)),
            pl.BlockSpec((f, bB), lambda i: (0, i)),
            full(bias11),
            full(W1), full(b1), full(g1), full(be1),
            full(W2), full(b2), full(g2), full(be2),
            full(W3), full(b3), full(g3), full(be3),
            full(Wo), full(bo11),
        ],
        out_specs=pl.BlockSpec((1, bB), lambda i: (0, i)),
        out_shape=jax.ShapeDtypeStruct((1, b_total), jnp.float32),
    )(e_t, dx_t, fcv_t, bias11,
      W1, b1, g1, be1, W2, b2, g2, be2, W3, b3, g3, be3, Wo, bo11)
    return out.reshape(b_total)


def kernel(sparse_x, dense_x, emb, fc_w, bias,
           W1, b1, g1, be1, W2, b2, g2, be2, W3, b3, g3, be3, Wo, bo):
    b, f = sparse_x.shape
    _, v, d = emb.shape
    cb = b // _CHUNK

    emb_t = emb.transpose(0, 2, 1).reshape(f * d, v)
    fc_flat = fc_w.reshape(1, f * v)
    sx_t = sparse_x.T.astype(jnp.int32)
    offs = (jnp.arange(f, dtype=jnp.int32) * v)[:, None]
    xoff_t = (sx_t + offs).reshape(f * cb, _CHUNK)
    sx_t3 = sx_t.reshape(f, cb, _CHUNK)

    e_t, fcv = _sc_gather_t(emb_t, fc_flat, sx_t3, xoff_t)

    fcv_t = fcv.reshape(f, b)
    dx_t = dense_x.T
    col = lambda a: a.reshape(-1, 1)
    out = _tc_forward_t(e_t, dx_t, fcv_t, bias.reshape(1, 1),
                        W1, col(b1), col(g1), col(be1),
                        W2, col(b2), col(g2), col(be2),
                        W3, col(b3), col(g3), col(be3),
                        Wo, bo.reshape(1, 1), 1e-5)
    return out
